# baseline (speedup 1.0000x reference)
import functools

import jax
import jax.numpy as jnp
from jax import lax
from jax.experimental import pallas as pl
from jax.experimental.pallas import tpu as pltpu
from jax.experimental.pallas import tpu_sc as plsc

N = 10000
NPAD = 10240
E = 320000
H = 256
Q = 64
NS = 16
K = 80
RING = 4
NCHUNK = 252
E_PAD = NS * NCHUNK * K
ROWS = NPAD // NS
BLK = 2000
GRID = N // BLK


def _mm(x, w):
    return lax.dot_general(x, w, (((1,), (1,)), ((), ())),
                           preferred_element_type=jnp.float32)


def _split_quarters(m, m4_ref):
    for c in range(2):
        for p in range(2):
            q = 2 * c + p
            m4_ref[c, p] = m[:, q * Q:(q + 1) * Q]


def _emb_body(h_ref, w0_ref, b0_ref, w1_ref, b1_ref, mw_ref, mb_ref,
              feat_ref, m4_ref):
    x = jax.nn.relu(_mm(h_ref[...], w0_ref[...]) + b0_ref[...])
    x = jax.nn.relu(_mm(x, w1_ref[...]) + b1_ref[...])
    feat_ref[...] = x
    m = _mm(x, mw_ref[...]) + mb_ref[...]
    _split_quarters(m, m4_ref)


def _gru_math(a4_ref, feat_ref, wih_ref, whh_ref, bih_ref, bhh_ref):
    a = jnp.concatenate(
        [a4_ref[0, 0], a4_ref[0, 1], a4_ref[1, 0], a4_ref[1, 1]], axis=1)
    gi = _mm(a, wih_ref[...]) + bih_ref[...]
    feat = feat_ref[...]
    gh = _mm(feat, whh_ref[...]) + bhh_ref[...]
    r = jax.nn.sigmoid(gi[:, :H] + gh[:, :H])
    z = jax.nn.sigmoid(gi[:, H:2 * H] + gh[:, H:2 * H])
    n = jnp.tanh(gi[:, 2 * H:] + r * gh[:, 2 * H:])
    return (1.0 - z) * n + z * feat


def _gru_msg_body(a4_ref, feat_ref, wih_ref, whh_ref, bih_ref, bhh_ref,
                  mw_ref, mb_ref, nfeat_ref, m4_ref):
    f = _gru_math(a4_ref, feat_ref, wih_ref, whh_ref, bih_ref, bhh_ref)
    nfeat_ref[...] = f
    m = _mm(f, mw_ref[...]) + mb_ref[...]
    _split_quarters(m, m4_ref)


def _gru_dec_body(a4_ref, feat_ref, wih_ref, whh_ref, bih_ref, bhh_ref,
                  dw0_ref, db0_ref, dw1_ref, db1_ref, dw2_ref, db2_ref,
                  out_ref):
    f = _gru_math(a4_ref, feat_ref, wih_ref, whh_ref, bih_ref, bhh_ref)
    x = jax.nn.relu(f)
    x = jax.nn.relu(_mm(x, dw0_ref[...]) + db0_ref[...])
    x = jax.nn.relu(_mm(x, dw1_ref[...]) + db1_ref[...])
    out_ref[...] = _mm(x, dw2_ref[...]) + db2_ref[...]


def _full(shape):
    return pl.BlockSpec(shape, lambda i: (0,) * len(shape))


_M4_SPEC = pl.BlockSpec((2, 2, BLK, Q), lambda i: (0, 0, i, 0))

_emb_call = pl.pallas_call(
    _emb_body,
    grid=(GRID,),
    in_specs=[
        pl.BlockSpec((BLK, 128), lambda i: (i, 0)),
        _full((128, 128)), _full((1, 128)),
        _full((256, 128)), _full((1, 256)),
        _full((256, 256)), _full((1, 256)),
    ],
    out_specs=[
        pl.BlockSpec((BLK, 256), lambda i: (i, 0)),
        _M4_SPEC,
    ],
    out_shape=[
        jax.ShapeDtypeStruct((N, 256), jnp.float32),
        jax.ShapeDtypeStruct((2, 2, N, Q), jnp.float32),
    ],
)

_gru_msg_call = pl.pallas_call(
    _gru_msg_body,
    grid=(GRID,),
    in_specs=[
        _M4_SPEC,
        pl.BlockSpec((BLK, 256), lambda i: (i, 0)),
        _full((768, 256)), _full((768, 256)),
        _full((1, 768)), _full((1, 768)),
        _full((256, 256)), _full((1, 256)),
    ],
    out_specs=[
        pl.BlockSpec((BLK, 256), lambda i: (i, 0)),
        _M4_SPEC,
    ],
    out_shape=[
        jax.ShapeDtypeStruct((N, 256), jnp.float32),
        jax.ShapeDtypeStruct((2, 2, N, Q), jnp.float32),
    ],
)

_gru_dec_call = pl.pallas_call(
    _gru_dec_body,
    grid=(GRID,),
    in_specs=[
        _M4_SPEC,
        pl.BlockSpec((BLK, 256), lambda i: (i, 0)),
        _full((768, 256)), _full((768, 256)),
        _full((1, 768)), _full((1, 768)),
        _full((128, 256)), _full((1, 128)),
        _full((64, 128)), _full((1, 64)),
        _full((64, 64)), _full((1, 64)),
    ],
    out_specs=[pl.BlockSpec((BLK, 64), lambda i: (i, 0))],
    out_shape=[jax.ShapeDtypeStruct((N, 64), jnp.float32)],
)


def _segsum_body(m4_hbm, edges_hbm, zeros_hbm, out_hbm,
                 src_v, dst_v, *rest):
    rbs = rest[:RING]
    acc = rest[RING]
    gsem = rest[RING + 1:RING + 1 + RING]
    ssem = rest[RING + 1 + RING:]
    c = lax.axis_index("c")
    s = lax.axis_index("s")
    pltpu.sync_copy(edges_hbm.at[0, s], src_v)
    pltpu.sync_copy(edges_hbm.at[1, s], dst_v)
    r0 = s * ROWS
    for p in range(2):
        pltpu.sync_copy(zeros_hbm, acc.at[pl.ds(r0, ROWS)])
        plsc.subcore_barrier()
        table = m4_hbm.at[c, p]
        pltpu.async_copy(table.at[src_v.at[0]], rbs[0], gsem[0])
        pltpu.async_copy(table.at[src_v.at[1]], rbs[1], gsem[1])

        def body(i, carry):
            j0 = RING * i
            for b in range(RING):
                j = j0 + b
                pltpu.make_async_copy(
                    table.at[src_v.at[j]], rbs[b], gsem[b]).wait()
                pltpu.async_copy(
                    rbs[b], acc.at[dst_v.at[j]], ssem[b], add=True)
                b2 = (b + 2) % RING

                @pl.when(j >= 2)
                def _():
                    pltpu.make_async_copy(
                        rbs[b2], acc.at[dst_v.at[j - 2]], ssem[b2]).wait()

                @pl.when(j + 2 < NCHUNK)
                def _():
                    pltpu.async_copy(
                        table.at[src_v.at[j + 2]], rbs[b2], gsem[b2])
            return carry

        lax.fori_loop(0, NCHUNK // RING, body, 0)
        pltpu.make_async_copy(
            rbs[(NCHUNK - 2) % RING],
            acc.at[dst_v.at[NCHUNK - 2]],
            ssem[(NCHUNK - 2) % RING]).wait()
        pltpu.make_async_copy(
            rbs[(NCHUNK - 1) % RING],
            acc.at[dst_v.at[NCHUNK - 1]],
            ssem[(NCHUNK - 1) % RING]).wait()
        plsc.subcore_barrier()
        pltpu.sync_copy(acc.at[pl.ds(r0, ROWS)],
                        out_hbm.at[c, p, pl.ds(r0, ROWS)])
        plsc.subcore_barrier()


@functools.cache
def _segsum_call():
    return pl.kernel(
        _segsum_body,
        out_type=jax.ShapeDtypeStruct((2, 2, NPAD, Q), jnp.float32),
        mesh=plsc.VectorSubcoreMesh(core_axis_name="c", subcore_axis_name="s",
                                    num_cores=2, num_subcores=NS),
        compiler_params=pltpu.CompilerParams(use_tc_tiling_on_sc=False),
        scratch_types=(
            [pltpu.VMEM((NCHUNK, K), jnp.int32)] * 2
            + [pltpu.VMEM((K, Q), jnp.float32)] * RING
            + [pltpu.VMEM_SHARED((NPAD, Q), jnp.float32)]
            + [pltpu.SemaphoreType.DMA] * (2 * RING)
        ),
    )


def kernel(h, edge_index, emb_W0, emb_b0, emb_W1, emb_b1, msg_W, msg_b,
           gru_W_ih, gru_W_hh, gru_b_ih, gru_b_hh,
           dec_W0, dec_b0, dec_W1, dec_b1, dec_W2, dec_b2):
    pad = E_PAD - E
    edges = jnp.concatenate(
        [edge_index,
         jnp.stack([jnp.zeros((pad,), jnp.int32),
                    jnp.full((pad,), NPAD - 1, jnp.int32)])], axis=1)
    edges = edges.reshape(2, NS, NCHUNK, K)
    zeros = jnp.zeros((ROWS, Q), jnp.float32)
    b = lambda v: v.reshape(1, -1)

    feat, m4 = _emb_call(h, emb_W0, b(emb_b0), emb_W1, b(emb_b1),
                         msg_W, b(msg_b))
    segsum = _segsum_call()
    for _ in range(2):
        a4 = segsum(m4, edges, zeros)
        feat, m4 = _gru_msg_call(a4, feat, gru_W_ih, gru_W_hh,
                                 b(gru_b_ih), b(gru_b_hh), msg_W, b(msg_b))
    a4 = segsum(m4, edges, zeros)
    (out,) = _gru_dec_call(a4, feat, gru_W_ih, gru_W_hh,
                           b(gru_b_ih), b(gru_b_hh),
                           dec_W0, b(dec_b0), dec_W1, b(dec_b1),
                           dec_W2, b(dec_b2))
    return out


# speedup vs baseline: 1.0430x; 1.0430x over previous
import functools

import jax
import jax.numpy as jnp
from jax import lax
from jax.experimental import pallas as pl
from jax.experimental.pallas import tpu as pltpu
from jax.experimental.pallas import tpu_sc as plsc

N = 10000
NPAD = 10240
E = 320000
H = 256
Q = 64
NS = 16
K = 80
RING = 3
NCHUNK = 252
E_PAD = NS * NCHUNK * K
ROWS = NPAD // NS
BLK = 2000
GRID = N // BLK


def _mm(x, w):
    return lax.dot_general(x, w, (((1,), (1,)), ((), ())),
                           preferred_element_type=jnp.float32)


def _split_quarters(m, m4_ref):
    for c in range(2):
        for p in range(2):
            q = 2 * c + p
            m4_ref[c, p] = m[:, q * Q:(q + 1) * Q]


def _emb_body(h_ref, w0_ref, b0_ref, w1_ref, b1_ref, mw_ref, mb_ref,
              feat_ref, m4_ref):
    x = jax.nn.relu(_mm(h_ref[...], w0_ref[...]) + b0_ref[...])
    x = jax.nn.relu(_mm(x, w1_ref[...]) + b1_ref[...])
    feat_ref[...] = x
    m = _mm(x, mw_ref[...]) + mb_ref[...]
    _split_quarters(m, m4_ref)


def _gru_math(a4_ref, feat_ref, wih_ref, whh_ref, bih_ref, bhh_ref):
    a = jnp.concatenate(
        [a4_ref[0, 0], a4_ref[0, 1], a4_ref[1, 0], a4_ref[1, 1]], axis=1)
    gi = _mm(a, wih_ref[...]) + bih_ref[...]
    feat = feat_ref[...]
    gh = _mm(feat, whh_ref[...]) + bhh_ref[...]
    r = jax.nn.sigmoid(gi[:, :H] + gh[:, :H])
    z = jax.nn.sigmoid(gi[:, H:2 * H] + gh[:, H:2 * H])
    n = jnp.tanh(gi[:, 2 * H:] + r * gh[:, 2 * H:])
    return (1.0 - z) * n + z * feat


def _gru_msg_body(a4_ref, feat_ref, wih_ref, whh_ref, bih_ref, bhh_ref,
                  mw_ref, mb_ref, nfeat_ref, m4_ref):
    f = _gru_math(a4_ref, feat_ref, wih_ref, whh_ref, bih_ref, bhh_ref)
    nfeat_ref[...] = f
    m = _mm(f, mw_ref[...]) + mb_ref[...]
    _split_quarters(m, m4_ref)


def _gru_dec_body(a4_ref, feat_ref, wih_ref, whh_ref, bih_ref, bhh_ref,
                  dw0_ref, db0_ref, dw1_ref, db1_ref, dw2_ref, db2_ref,
                  out_ref):
    f = _gru_math(a4_ref, feat_ref, wih_ref, whh_ref, bih_ref, bhh_ref)
    x = jax.nn.relu(f)
    x = jax.nn.relu(_mm(x, dw0_ref[...]) + db0_ref[...])
    x = jax.nn.relu(_mm(x, dw1_ref[...]) + db1_ref[...])
    out_ref[...] = _mm(x, dw2_ref[...]) + db2_ref[...]


def _full(shape):
    return pl.BlockSpec(shape, lambda i: (0,) * len(shape))


_M4_SPEC = pl.BlockSpec((2, 2, BLK, Q), lambda i: (0, 0, i, 0))

_emb_call = pl.pallas_call(
    _emb_body,
    grid=(GRID,),
    in_specs=[
        pl.BlockSpec((BLK, 128), lambda i: (i, 0)),
        _full((128, 128)), _full((1, 128)),
        _full((256, 128)), _full((1, 256)),
        _full((256, 256)), _full((1, 256)),
    ],
    out_specs=[
        pl.BlockSpec((BLK, 256), lambda i: (i, 0)),
        _M4_SPEC,
    ],
    out_shape=[
        jax.ShapeDtypeStruct((N, 256), jnp.float32),
        jax.ShapeDtypeStruct((2, 2, N, Q), jnp.float32),
    ],
)

_gru_msg_call = pl.pallas_call(
    _gru_msg_body,
    grid=(GRID,),
    in_specs=[
        _M4_SPEC,
        pl.BlockSpec((BLK, 256), lambda i: (i, 0)),
        _full((768, 256)), _full((768, 256)),
        _full((1, 768)), _full((1, 768)),
        _full((256, 256)), _full((1, 256)),
    ],
    out_specs=[
        pl.BlockSpec((BLK, 256), lambda i: (i, 0)),
        _M4_SPEC,
    ],
    out_shape=[
        jax.ShapeDtypeStruct((N, 256), jnp.float32),
        jax.ShapeDtypeStruct((2, 2, N, Q), jnp.float32),
    ],
)

_gru_dec_call = pl.pallas_call(
    _gru_dec_body,
    grid=(GRID,),
    in_specs=[
        _M4_SPEC,
        pl.BlockSpec((BLK, 256), lambda i: (i, 0)),
        _full((768, 256)), _full((768, 256)),
        _full((1, 768)), _full((1, 768)),
        _full((128, 256)), _full((1, 128)),
        _full((64, 128)), _full((1, 64)),
        _full((64, 64)), _full((1, 64)),
    ],
    out_specs=[pl.BlockSpec((BLK, 64), lambda i: (i, 0))],
    out_shape=[jax.ShapeDtypeStruct((N, 64), jnp.float32)],
)


def _segsum_body(m4_hbm, edges_hbm, zeros_hbm, out_hbm,
                 src_v, dst_v, *rest):
    rbs = rest[:RING]
    acc = rest[RING]
    gsem = rest[RING + 1:RING + 1 + RING]
    ssem = rest[RING + 1 + RING:]
    c = lax.axis_index("c")
    s = lax.axis_index("s")
    pltpu.sync_copy(edges_hbm.at[0, s], src_v)
    pltpu.sync_copy(edges_hbm.at[1, s], dst_v)
    r0 = s * ROWS
    for p in range(2):
        pltpu.sync_copy(zeros_hbm, acc.at[pl.ds(r0, ROWS)])
        plsc.subcore_barrier()
        table = m4_hbm.at[c, p]
        pltpu.async_copy(table.at[src_v.at[0]], rbs[0], gsem[0])
        pltpu.async_copy(table.at[src_v.at[1]], rbs[1], gsem[1])

        def body(i, carry):
            j0 = RING * i
            for b in range(RING):
                j = j0 + b
                pltpu.make_async_copy(
                    table.at[src_v.at[j]], rbs[b], gsem[b]).wait()
                b2 = (b + 2) % RING

                @pl.when(j + 2 < NCHUNK)
                def _():
                    pltpu.async_copy(
                        table.at[src_v.at[j + 2]], rbs[b2], gsem[b2])

                pltpu.sync_copy(rbs[b], acc.at[dst_v.at[j]], add=True)
            return carry

        lax.fori_loop(0, NCHUNK // RING, body, 0)
        plsc.subcore_barrier()
        pltpu.sync_copy(acc.at[pl.ds(r0, ROWS)],
                        out_hbm.at[c, p, pl.ds(r0, ROWS)])
        plsc.subcore_barrier()


@functools.cache
def _segsum_call():
    return pl.kernel(
        _segsum_body,
        out_type=jax.ShapeDtypeStruct((2, 2, NPAD, Q), jnp.float32),
        mesh=plsc.VectorSubcoreMesh(core_axis_name="c", subcore_axis_name="s",
                                    num_cores=2, num_subcores=NS),
        compiler_params=pltpu.CompilerParams(use_tc_tiling_on_sc=False),
        scratch_types=(
            [pltpu.VMEM((NCHUNK, K), jnp.int32)] * 2
            + [pltpu.VMEM((K, Q), jnp.float32)] * RING
            + [pltpu.VMEM_SHARED((NPAD, Q), jnp.float32)]
            + [pltpu.SemaphoreType.DMA] * (2 * RING)
        ),
    )


def kernel(h, edge_index, emb_W0, emb_b0, emb_W1, emb_b1, msg_W, msg_b,
           gru_W_ih, gru_W_hh, gru_b_ih, gru_b_hh,
           dec_W0, dec_b0, dec_W1, dec_b1, dec_W2, dec_b2):
    pad = E_PAD - E
    edges = jnp.concatenate(
        [edge_index,
         jnp.stack([jnp.zeros((pad,), jnp.int32),
                    jnp.full((pad,), NPAD - 1, jnp.int32)])], axis=1)
    edges = edges.reshape(2, NS, NCHUNK, K)
    zeros = jnp.zeros((ROWS, Q), jnp.float32)
    b = lambda v: v.reshape(1, -1)

    feat, m4 = _emb_call(h, emb_W0, b(emb_b0), emb_W1, b(emb_b1),
                         msg_W, b(msg_b))
    segsum = _segsum_call()
    for _ in range(2):
        a4 = segsum(m4, edges, zeros)
        feat, m4 = _gru_msg_call(a4, feat, gru_W_ih, gru_W_hh,
                                 b(gru_b_ih), b(gru_b_hh), msg_W, b(msg_b))
    a4 = segsum(m4, edges, zeros)
    (out,) = _gru_dec_call(a4, feat, gru_W_ih, gru_W_hh,
                           b(gru_b_ih), b(gru_b_hh),
                           dec_W0, b(dec_b0), dec_W1, b(dec_b1),
                           dec_W2, b(dec_b2))
    return out


# speedup vs baseline: 1.2204x; 1.1701x over previous
import functools

import jax
import jax.numpy as jnp
from jax import lax
from jax.experimental import pallas as pl
from jax.experimental.pallas import tpu as pltpu
from jax.experimental.pallas import tpu_sc as plsc

N = 10000
NPAD = 10240
E = 320000
H = 256
Q = 64
NS = 16
K = 80
RING = 2
NCHUNK = E // NS // K
E_PAD = NS * NCHUNK * K
ROWS = NPAD // NS
BLK = 2000
GRID = N // BLK


def _mm(x, w):
    return lax.dot_general(x, w, (((1,), (1,)), ((), ())),
                           preferred_element_type=jnp.float32)


def _split_quarters(m, m4_ref):
    for c in range(2):
        for p in range(2):
            q = 2 * c + p
            m4_ref[c, p] = m[:, q * Q:(q + 1) * Q]


def _emb_body(h_ref, w0_ref, b0_ref, w1_ref, b1_ref, mw_ref, mb_ref,
              feat_ref, m4_ref):
    x = jax.nn.relu(_mm(h_ref[...], w0_ref[...]) + b0_ref[...])
    x = jax.nn.relu(_mm(x, w1_ref[...]) + b1_ref[...])
    feat_ref[...] = x
    m = _mm(x, mw_ref[...]) + mb_ref[...]
    _split_quarters(m, m4_ref)


def _gru_math(a4_ref, feat_ref, wih_ref, whh_ref, bih_ref, bhh_ref):
    a = jnp.concatenate(
        [a4_ref[0, 0], a4_ref[0, 1], a4_ref[1, 0], a4_ref[1, 1]], axis=1)
    gi = _mm(a, wih_ref[...]) + bih_ref[...]
    feat = feat_ref[...]
    gh = _mm(feat, whh_ref[...]) + bhh_ref[...]
    r = jax.nn.sigmoid(gi[:, :H] + gh[:, :H])
    z = jax.nn.sigmoid(gi[:, H:2 * H] + gh[:, H:2 * H])
    n = jnp.tanh(gi[:, 2 * H:] + r * gh[:, 2 * H:])
    return (1.0 - z) * n + z * feat


def _gru_msg_body(a4_ref, feat_ref, wih_ref, whh_ref, bih_ref, bhh_ref,
                  mw_ref, mb_ref, nfeat_ref, m4_ref):
    f = _gru_math(a4_ref, feat_ref, wih_ref, whh_ref, bih_ref, bhh_ref)
    nfeat_ref[...] = f
    m = _mm(f, mw_ref[...]) + mb_ref[...]
    _split_quarters(m, m4_ref)


def _gru_dec_body(a4_ref, feat_ref, wih_ref, whh_ref, bih_ref, bhh_ref,
                  dw0_ref, db0_ref, dw1_ref, db1_ref, dw2_ref, db2_ref,
                  out_ref):
    f = _gru_math(a4_ref, feat_ref, wih_ref, whh_ref, bih_ref, bhh_ref)
    x = jax.nn.relu(f)
    x = jax.nn.relu(_mm(x, dw0_ref[...]) + db0_ref[...])
    x = jax.nn.relu(_mm(x, dw1_ref[...]) + db1_ref[...])
    out_ref[...] = _mm(x, dw2_ref[...]) + db2_ref[...]


def _full(shape):
    return pl.BlockSpec(shape, lambda i: (0,) * len(shape))


_M4_SPEC = pl.BlockSpec((2, 2, BLK, Q), lambda i: (0, 0, i, 0))

_emb_call = pl.pallas_call(
    _emb_body,
    grid=(GRID,),
    in_specs=[
        pl.BlockSpec((BLK, 128), lambda i: (i, 0)),
        _full((128, 128)), _full((1, 128)),
        _full((256, 128)), _full((1, 256)),
        _full((256, 256)), _full((1, 256)),
    ],
    out_specs=[
        pl.BlockSpec((BLK, 256), lambda i: (i, 0)),
        _M4_SPEC,
    ],
    out_shape=[
        jax.ShapeDtypeStruct((N, 256), jnp.float32),
        jax.ShapeDtypeStruct((2, 2, N, Q), jnp.float32),
    ],
)

_gru_msg_call = pl.pallas_call(
    _gru_msg_body,
    grid=(GRID,),
    in_specs=[
        _M4_SPEC,
        pl.BlockSpec((BLK, 256), lambda i: (i, 0)),
        _full((768, 256)), _full((768, 256)),
        _full((1, 768)), _full((1, 768)),
        _full((256, 256)), _full((1, 256)),
    ],
    out_specs=[
        pl.BlockSpec((BLK, 256), lambda i: (i, 0)),
        _M4_SPEC,
    ],
    out_shape=[
        jax.ShapeDtypeStruct((N, 256), jnp.float32),
        jax.ShapeDtypeStruct((2, 2, N, Q), jnp.float32),
    ],
)

_gru_dec_call = pl.pallas_call(
    _gru_dec_body,
    grid=(GRID,),
    in_specs=[
        _M4_SPEC,
        pl.BlockSpec((BLK, 256), lambda i: (i, 0)),
        _full((768, 256)), _full((768, 256)),
        _full((1, 768)), _full((1, 768)),
        _full((128, 256)), _full((1, 128)),
        _full((64, 128)), _full((1, 64)),
        _full((64, 64)), _full((1, 64)),
    ],
    out_specs=[pl.BlockSpec((BLK, 64), lambda i: (i, 0))],
    out_shape=[jax.ShapeDtypeStruct((N, 64), jnp.float32)],
)


def _segsum_body(m4_hbm, edges_hbm, zeros_hbm, out_hbm,
                 src_v, dst_v, *rest):
    rbs = rest[:RING]
    acc = rest[RING]
    gsem = rest[RING + 1:RING + 1 + RING]
    ssem = rest[RING + 1 + RING:]
    c = lax.axis_index("c")
    s = lax.axis_index("s")
    pltpu.sync_copy(edges_hbm.at[0, s], src_v)
    pltpu.sync_copy(edges_hbm.at[1, s], dst_v)
    r0 = s * ROWS
    for p in range(2):
        pltpu.sync_copy(zeros_hbm, acc.at[pl.ds(r0, ROWS)])
        plsc.subcore_barrier()
        table = m4_hbm.at[c, p]
        rb0, rb1 = rbs[0], rbs[1]
        sem0, sem1 = gsem[0], gsem[1]
        pltpu.async_copy(table.at[src_v.at[0]], rb0, sem0)

        def body(i, carry):
            j0 = 2 * i
            pltpu.async_copy(table.at[src_v.at[j0 + 1]], rb1, sem1)
            pltpu.make_async_copy(table.at[src_v.at[j0]], rb0, sem0).wait()
            pltpu.sync_copy(rb0, acc.at[dst_v.at[j0]], add=True)

            @pl.when(j0 + 2 < NCHUNK)
            def _():
                pltpu.async_copy(table.at[src_v.at[j0 + 2]], rb0, sem0)

            pltpu.make_async_copy(table.at[src_v.at[j0 + 1]], rb1, sem1).wait()
            pltpu.sync_copy(rb1, acc.at[dst_v.at[j0 + 1]], add=True)
            return carry

        lax.fori_loop(0, NCHUNK // 2, body, 0)
        plsc.subcore_barrier()
        pltpu.sync_copy(acc.at[pl.ds(r0, ROWS)],
                        out_hbm.at[c, p, pl.ds(r0, ROWS)])
        plsc.subcore_barrier()


@functools.cache
def _segsum_call():
    return pl.kernel(
        _segsum_body,
        out_type=jax.ShapeDtypeStruct((2, 2, NPAD, Q), jnp.float32),
        mesh=plsc.VectorSubcoreMesh(core_axis_name="c", subcore_axis_name="s",
                                    num_cores=2, num_subcores=NS),
        compiler_params=pltpu.CompilerParams(use_tc_tiling_on_sc=False),
        scratch_types=(
            [pltpu.VMEM((NCHUNK, K), jnp.int32)] * 2
            + [pltpu.VMEM((K, Q), jnp.float32)] * RING
            + [pltpu.VMEM_SHARED((NPAD, Q), jnp.float32)]
            + [pltpu.SemaphoreType.DMA] * (2 * RING)
        ),
    )


def kernel(h, edge_index, emb_W0, emb_b0, emb_W1, emb_b1, msg_W, msg_b,
           gru_W_ih, gru_W_hh, gru_b_ih, gru_b_hh,
           dec_W0, dec_b0, dec_W1, dec_b1, dec_W2, dec_b2):
    pad = E_PAD - E
    if pad:
        edge_index = jnp.concatenate(
            [edge_index,
             jnp.stack([jnp.zeros((pad,), jnp.int32),
                        jnp.full((pad,), NPAD - 1, jnp.int32)])], axis=1)
    edges = edge_index.reshape(2, NS, NCHUNK, K)
    zeros = jnp.zeros((ROWS, Q), jnp.float32)
    b = lambda v: v.reshape(1, -1)

    feat, m4 = _emb_call(h, emb_W0, b(emb_b0), emb_W1, b(emb_b1),
                         msg_W, b(msg_b))
    segsum = _segsum_call()
    for _ in range(2):
        a4 = segsum(m4, edges, zeros)
        feat, m4 = _gru_msg_call(a4, feat, gru_W_ih, gru_W_hh,
                                 b(gru_b_ih), b(gru_b_hh), msg_W, b(msg_b))
    a4 = segsum(m4, edges, zeros)
    (out,) = _gru_dec_call(a4, feat, gru_W_ih, gru_W_hh,
                           b(gru_b_ih), b(gru_b_hh),
                           dec_W0, b(dec_b0), dec_W1, b(dec_b1),
                           dec_W2, b(dec_b2))
    return out


# speedup vs baseline: 1.4286x; 1.1706x over previous
import functools

import jax
import jax.numpy as jnp
from jax import lax
from jax.experimental import pallas as pl
from jax.experimental.pallas import tpu as pltpu
from jax.experimental.pallas import tpu_sc as plsc

N = 10000
NPAD = 10240
E = 320000
H = 256
Q = 64
NS = 16
K = 125
RING = 2
NCHUNK = E // NS // K
E_PAD = NS * NCHUNK * K
ROWS = NPAD // NS
BLK = 2000
GRID = N // BLK


def _mm(x, w):
    return lax.dot_general(x, w, (((1,), (1,)), ((), ())),
                           preferred_element_type=jnp.float32)


def _split_quarters(m, m4_ref):
    for c in range(2):
        for p in range(2):
            q = 2 * c + p
            m4_ref[c, p] = m[:, q * Q:(q + 1) * Q]


def _emb_body(h_ref, w0_ref, b0_ref, w1_ref, b1_ref, mw_ref, mb_ref,
              feat_ref, m4_ref):
    x = jax.nn.relu(_mm(h_ref[...], w0_ref[...]) + b0_ref[...])
    x = jax.nn.relu(_mm(x, w1_ref[...]) + b1_ref[...])
    feat_ref[...] = x
    m = _mm(x, mw_ref[...]) + mb_ref[...]
    _split_quarters(m, m4_ref)


def _gru_math(a4_ref, feat_ref, wih_ref, whh_ref, bih_ref, bhh_ref):
    a = jnp.concatenate(
        [a4_ref[0, 0], a4_ref[0, 1], a4_ref[1, 0], a4_ref[1, 1]], axis=1)
    gi = _mm(a, wih_ref[...]) + bih_ref[...]
    feat = feat_ref[...]
    gh = _mm(feat, whh_ref[...]) + bhh_ref[...]
    r = jax.nn.sigmoid(gi[:, :H] + gh[:, :H])
    z = jax.nn.sigmoid(gi[:, H:2 * H] + gh[:, H:2 * H])
    n = jnp.tanh(gi[:, 2 * H:] + r * gh[:, 2 * H:])
    return (1.0 - z) * n + z * feat


def _gru_msg_body(a4_ref, feat_ref, wih_ref, whh_ref, bih_ref, bhh_ref,
                  mw_ref, mb_ref, nfeat_ref, m4_ref):
    f = _gru_math(a4_ref, feat_ref, wih_ref, whh_ref, bih_ref, bhh_ref)
    nfeat_ref[...] = f
    m = _mm(f, mw_ref[...]) + mb_ref[...]
    _split_quarters(m, m4_ref)


def _gru_dec_body(a4_ref, feat_ref, wih_ref, whh_ref, bih_ref, bhh_ref,
                  dw0_ref, db0_ref, dw1_ref, db1_ref, dw2_ref, db2_ref,
                  out_ref):
    f = _gru_math(a4_ref, feat_ref, wih_ref, whh_ref, bih_ref, bhh_ref)
    x = jax.nn.relu(f)
    x = jax.nn.relu(_mm(x, dw0_ref[...]) + db0_ref[...])
    x = jax.nn.relu(_mm(x, dw1_ref[...]) + db1_ref[...])
    out_ref[...] = _mm(x, dw2_ref[...]) + db2_ref[...]


def _full(shape):
    return pl.BlockSpec(shape, lambda i: (0,) * len(shape))


_M4_SPEC = pl.BlockSpec((2, 2, BLK, Q), lambda i: (0, 0, i, 0))

_emb_call = pl.pallas_call(
    _emb_body,
    grid=(GRID,),
    in_specs=[
        pl.BlockSpec((BLK, 128), lambda i: (i, 0)),
        _full((128, 128)), _full((1, 128)),
        _full((256, 128)), _full((1, 256)),
        _full((256, 256)), _full((1, 256)),
    ],
    out_specs=[
        pl.BlockSpec((BLK, 256), lambda i: (i, 0)),
        _M4_SPEC,
    ],
    out_shape=[
        jax.ShapeDtypeStruct((N, 256), jnp.float32),
        jax.ShapeDtypeStruct((2, 2, N, Q), jnp.float32),
    ],
)

_gru_msg_call = pl.pallas_call(
    _gru_msg_body,
    grid=(GRID,),
    in_specs=[
        _M4_SPEC,
        pl.BlockSpec((BLK, 256), lambda i: (i, 0)),
        _full((768, 256)), _full((768, 256)),
        _full((1, 768)), _full((1, 768)),
        _full((256, 256)), _full((1, 256)),
    ],
    out_specs=[
        pl.BlockSpec((BLK, 256), lambda i: (i, 0)),
        _M4_SPEC,
    ],
    out_shape=[
        jax.ShapeDtypeStruct((N, 256), jnp.float32),
        jax.ShapeDtypeStruct((2, 2, N, Q), jnp.float32),
    ],
)

_gru_dec_call = pl.pallas_call(
    _gru_dec_body,
    grid=(GRID,),
    in_specs=[
        _M4_SPEC,
        pl.BlockSpec((BLK, 256), lambda i: (i, 0)),
        _full((768, 256)), _full((768, 256)),
        _full((1, 768)), _full((1, 768)),
        _full((128, 256)), _full((1, 128)),
        _full((64, 128)), _full((1, 64)),
        _full((64, 64)), _full((1, 64)),
    ],
    out_specs=[pl.BlockSpec((BLK, 64), lambda i: (i, 0))],
    out_shape=[jax.ShapeDtypeStruct((N, 64), jnp.float32)],
)


def _segsum_body(m4_hbm, edges_hbm, zeros_hbm, out_hbm,
                 src_v, dst_v, *rest):
    rbs = rest[:RING]
    acc = rest[RING]
    gsem = rest[RING + 1:RING + 1 + RING]
    ssem = rest[RING + 1 + RING:]
    c = lax.axis_index("c")
    s = lax.axis_index("s")
    pltpu.sync_copy(edges_hbm.at[0, s], src_v)
    pltpu.sync_copy(edges_hbm.at[1, s], dst_v)
    r0 = s * ROWS
    for p in range(2):
        pltpu.sync_copy(zeros_hbm, acc.at[pl.ds(r0, ROWS)])
        plsc.subcore_barrier()
        table = m4_hbm.at[c, p]
        rb0, rb1 = rbs[0], rbs[1]
        sem0, sem1 = gsem[0], gsem[1]
        pltpu.async_copy(table.at[src_v.at[0]], rb0, sem0)

        def body(i, carry):
            j0 = 2 * i
            pltpu.async_copy(table.at[src_v.at[j0 + 1]], rb1, sem1)
            pltpu.make_async_copy(table.at[src_v.at[j0]], rb0, sem0).wait()
            pltpu.sync_copy(rb0, acc.at[dst_v.at[j0]], add=True)

            @pl.when(j0 + 2 < NCHUNK)
            def _():
                pltpu.async_copy(table.at[src_v.at[j0 + 2]], rb0, sem0)

            pltpu.make_async_copy(table.at[src_v.at[j0 + 1]], rb1, sem1).wait()
            pltpu.sync_copy(rb1, acc.at[dst_v.at[j0 + 1]], add=True)
            return carry

        lax.fori_loop(0, NCHUNK // 2, body, 0)
        plsc.subcore_barrier()
        pltpu.sync_copy(acc.at[pl.ds(r0, ROWS)],
                        out_hbm.at[c, p, pl.ds(r0, ROWS)])
        plsc.subcore_barrier()


@functools.cache
def _segsum_call():
    return pl.kernel(
        _segsum_body,
        out_type=jax.ShapeDtypeStruct((2, 2, NPAD, Q), jnp.float32),
        mesh=plsc.VectorSubcoreMesh(core_axis_name="c", subcore_axis_name="s",
                                    num_cores=2, num_subcores=NS),
        compiler_params=pltpu.CompilerParams(use_tc_tiling_on_sc=False),
        scratch_types=(
            [pltpu.VMEM((NCHUNK, K), jnp.int32)] * 2
            + [pltpu.VMEM((K, Q), jnp.float32)] * RING
            + [pltpu.VMEM_SHARED((NPAD, Q), jnp.float32)]
            + [pltpu.SemaphoreType.DMA] * (2 * RING)
        ),
    )


def kernel(h, edge_index, emb_W0, emb_b0, emb_W1, emb_b1, msg_W, msg_b,
           gru_W_ih, gru_W_hh, gru_b_ih, gru_b_hh,
           dec_W0, dec_b0, dec_W1, dec_b1, dec_W2, dec_b2):
    pad = E_PAD - E
    if pad:
        edge_index = jnp.concatenate(
            [edge_index,
             jnp.stack([jnp.zeros((pad,), jnp.int32),
                        jnp.full((pad,), NPAD - 1, jnp.int32)])], axis=1)
    edges = edge_index.reshape(2, NS, NCHUNK, K)
    zeros = jnp.zeros((ROWS, Q), jnp.float32)
    b = lambda v: v.reshape(1, -1)

    feat, m4 = _emb_call(h, emb_W0, b(emb_b0), emb_W1, b(emb_b1),
                         msg_W, b(msg_b))
    segsum = _segsum_call()
    for _ in range(2):
        a4 = segsum(m4, edges, zeros)
        feat, m4 = _gru_msg_call(a4, feat, gru_W_ih, gru_W_hh,
                                 b(gru_b_ih), b(gru_b_hh), msg_W, b(msg_b))
    a4 = segsum(m4, edges, zeros)
    (out,) = _gru_dec_call(a4, feat, gru_W_ih, gru_W_hh,
                           b(gru_b_ih), b(gru_b_hh),
                           dec_W0, b(dec_b0), dec_W1, b(dec_b1),
                           dec_W2, b(dec_b2))
    return out
